# Initial kernel scaffold; baseline (speedup 1.0000x reference)
import jax
import jax.numpy as jnp
from jax.experimental import pallas as pl


def kernel(x, edge_index, fc0_w, fc0_b, bn0_g, bn0_b, wq_w, wq_b, wk_w, wk_b, wo_w, wo_b, bn1_g, bn1_b, fc1_w, fc1_b):
    raise NotImplementedError("write your pallas kernel here")


# speedup vs baseline: 5.5685x; 5.5685x over previous
import functools

import numpy as np

import jax
import jax.numpy as jnp
from jax import lax
from jax.experimental import pallas as pl
from jax.experimental.pallas import tpu as pltpu
from jax.experimental.pallas import tpu_sc as plsc

N = 10000
D = 128
NP = 10240
BETA = 0.5
BN_DIV = float(np.sqrt(np.float32(1.0 + 1e-5)))

NW = 32
CHUNK = 128
EC = 80
EPW = EC * CHUNK
EP = NW * EPW
GRP = 8
NGRP = EC // GRP
PAD_IDX = N

ROWS_PER_TILE = NP // 16


def _sc_mesh():
    return plsc.VectorSubcoreMesh(core_axis_name="c", subcore_axis_name="s")


def _hist_body(col_hbm, out_hbm, hist_sh, colv, onesv):
    c = lax.axis_index("c")
    s = lax.axis_index("s")
    wid = s * 2 + c

    def _fill(val, i, _):
        r = i // 8
        q = i % 8
        onesv[r, pl.ds(q * 16, 16)] = jnp.full((16,), val, jnp.float32)
        return 0

    lax.fori_loop(0, CHUNK * 8, functools.partial(_fill, 0.0), 0)
    for k in range(ROWS_PER_TILE // CHUNK):
        pltpu.sync_copy(onesv,
                        hist_sh.at[pl.ds(s * ROWS_PER_TILE + k * CHUNK, CHUNK)])
    lax.fori_loop(0, CHUNK * 8, functools.partial(_fill, 1.0), 0)
    plsc.subcore_barrier()

    pltpu.sync_copy(col_hbm.at[pl.ds(wid * EC, EC)], colv)

    def _scat(j, _):
        pltpu.sync_copy(onesv, hist_sh.at[colv.at[j]], add=True)
        return 0
    lax.fori_loop(0, EC, _scat, 0)

    plsc.subcore_barrier()
    pltpu.sync_copy(hist_sh.at[pl.ds(s * ROWS_PER_TILE, ROWS_PER_TILE)],
                    out_hbm.at[c].at[pl.ds(s * ROWS_PER_TILE, ROWS_PER_TILE)])


def _sc_hist(col_p):
    k = pl.kernel(
        _hist_body,
        out_type=jax.ShapeDtypeStruct((2, NP, D), jnp.float32),
        mesh=_sc_mesh(),
        scratch_types=[
            pltpu.VMEM_SHARED((NP, D), jnp.float32),
            pltpu.VMEM((EC, CHUNK), jnp.int32),
            pltpu.VMEM((CHUNK, D), jnp.float32),
        ],
    )
    return k(col_p)


def _scatter_body(u_hbm, row_hbm, col_hbm, acc_hbm,
                  acc_sh, rowv, colv, rbuf0, rbuf1, sem0, sem1):
    c = lax.axis_index("c")
    s = lax.axis_index("s")
    wid = s * 2 + c

    def _zfill(i, _):
        r = i // 8
        q = i % 8
        rbuf0[r, pl.ds(q * 16, 16)] = jnp.zeros((16,), jnp.float32)
        return 0
    lax.fori_loop(0, CHUNK * 8, _zfill, 0)

    for k in range(ROWS_PER_TILE // CHUNK):
        pltpu.sync_copy(rbuf0,
                        acc_sh.at[pl.ds(s * ROWS_PER_TILE + k * CHUNK, CHUNK)])
    plsc.subcore_barrier()

    def _group(g, _):
        base = wid * EC + g * GRP
        pltpu.sync_copy(row_hbm.at[pl.ds(base, GRP)], rowv)
        pltpu.sync_copy(col_hbm.at[pl.ds(base, GRP)], colv)
        pltpu.async_copy(u_hbm.at[rowv.at[0]], rbuf0, sem0)

        def _step(j, _):
            @pl.when(j % 2 == 0)
            def _():
                @pl.when(j + 1 < GRP)
                def _():
                    pltpu.async_copy(u_hbm.at[rowv.at[j + 1]], rbuf1, sem1)
                pltpu.make_async_copy(u_hbm.at[rowv.at[j]], rbuf0, sem0).wait()
                pltpu.sync_copy(rbuf0, acc_sh.at[colv.at[j]], add=True)

            @pl.when(j % 2 == 1)
            def _():
                @pl.when(j + 1 < GRP)
                def _():
                    pltpu.async_copy(u_hbm.at[rowv.at[j + 1]], rbuf0, sem0)
                pltpu.make_async_copy(u_hbm.at[rowv.at[j]], rbuf1, sem1).wait()
                pltpu.sync_copy(rbuf1, acc_sh.at[colv.at[j]], add=True)
            return 0
        lax.fori_loop(0, GRP, _step, 0)
        return 0
    lax.fori_loop(0, NGRP, _group, 0)

    plsc.subcore_barrier()
    pltpu.sync_copy(acc_sh.at[pl.ds(s * ROWS_PER_TILE, ROWS_PER_TILE)],
                    acc_hbm.at[c].at[pl.ds(s * ROWS_PER_TILE, ROWS_PER_TILE)])


def _sc_scatter(u, row_p, col_p):
    k = pl.kernel(
        _scatter_body,
        out_type=jax.ShapeDtypeStruct((2, NP, D), jnp.float32),
        mesh=_sc_mesh(),
        scratch_types=[
            pltpu.VMEM_SHARED((NP, D), jnp.float32),
            pltpu.VMEM((GRP, CHUNK), jnp.int32),
            pltpu.VMEM((GRP, CHUNK), jnp.int32),
            pltpu.VMEM((CHUNK, D), jnp.float32),
            pltpu.VMEM((CHUNK, D), jnp.float32),
            pltpu.SemaphoreType.DMA,
            pltpu.SemaphoreType.DMA,
        ],
    )
    return k(u, row_p, col_p)


BLK = 512
NBLK = NP // BLK


def _b1_body(x_ref, fc0w_ref, fc0b_ref, bn0g_ref, bn0b_ref,
             wqw_ref, wqb_ref, wkw_ref, wkb_ref,
             x0_ref, qs_ref, ks_ref, kvs_ref, vsum_ref):
    i = pl.program_id(0)

    @pl.when(i == 0)
    def _():
        kvs_ref[...] = jnp.zeros_like(kvs_ref)
        vsum_ref[...] = jnp.zeros_like(vsum_ref)

    xb = x_ref[...]
    h = jnp.dot(xb, fc0w_ref[...], preferred_element_type=jnp.float32)
    h = (h + fc0b_ref[...]) * bn0g_ref[...] / BN_DIV + bn0b_ref[...]
    h = jnp.maximum(h, 0.0)
    rows = i * BLK + lax.broadcasted_iota(jnp.int32, (BLK, 1), 0)
    mask = rows < N
    h = jnp.where(mask, h, 0.0)

    q = jnp.dot(h, wqw_ref[...], preferred_element_type=jnp.float32) + wqb_ref[...]
    kk = jnp.dot(h, wkw_ref[...], preferred_element_type=jnp.float32) + wkb_ref[...]
    qn = jnp.sqrt(jnp.sum(q * q, axis=1, keepdims=True))
    kn = jnp.sqrt(jnp.sum(kk * kk, axis=1, keepdims=True))
    qs = jnp.where(mask, q / jnp.broadcast_to(qn, (BLK, D)), 0.0)
    ks = jnp.where(mask, kk / jnp.broadcast_to(kn, (BLK, D)), 0.0)

    x0_ref[...] = h
    qs_ref[...] = qs
    ks_ref[...] = ks
    kvs_ref[...] += lax.dot_general(ks, h, (((0,), (0,)), ((), ())),
                                    preferred_element_type=jnp.float32)
    vsum_ref[...] += jnp.broadcast_to(jnp.sum(h, axis=0, keepdims=True), (8, D))


def _tc_b1(x_p, fc0_w, fc0_b, bn0_g, bn0_b, wq_w, wq_b, wk_w, wk_b):
    row_spec = pl.BlockSpec((BLK, D), lambda i: (i, 0))
    full = lambda shp: pl.BlockSpec(shp, lambda i: tuple(0 for _ in shp))
    vec = pl.BlockSpec((D,), lambda i: (0,))
    return pl.pallas_call(
        _b1_body,
        grid=(NBLK,),
        in_specs=[row_spec, full((D, D)), vec, vec, vec,
                  full((D, D)), vec, full((D, D)), vec],
        out_specs=[row_spec, row_spec, row_spec, full((D, D)), full((8, D))],
        out_shape=[jax.ShapeDtypeStruct((NP, D), jnp.float32),
                   jax.ShapeDtypeStruct((NP, D), jnp.float32),
                   jax.ShapeDtypeStruct((NP, D), jnp.float32),
                   jax.ShapeDtypeStruct((D, D), jnp.float32),
                   jax.ShapeDtypeStruct((8, D), jnp.float32)],
    )(x_p, fc0_w, fc0_b, bn0_g, bn0_b, wq_w, wq_b, wk_w, wk_b)


def _b2_body(qs_ref, x0_ref, kvs_ref, ksmat_ref, vsum_ref, hist_ref,
             a_ref, u0_ref, d2_ref):
    qs = qs_ref[...]
    hb = hist_ref[...]
    deg = hb[0, :, 0:1] + hb[1, :, 0:1]
    dis = jnp.where(deg > 0, lax.rsqrt(jnp.maximum(deg, 1e-30)), 0.0)
    d2 = jnp.broadcast_to(dis, (BLK, D))

    num = jnp.dot(qs, kvs_ref[...], preferred_element_type=jnp.float32)
    num = num + jnp.broadcast_to(vsum_ref[0:1, :], (BLK, D))
    den = jnp.dot(qs, ksmat_ref[...], preferred_element_type=jnp.float32)
    den = den[:, 0:1] + 1.0
    a_ref[...] = num / jnp.broadcast_to(den, (BLK, D))
    d2_ref[...] = d2
    u0_ref[...] = d2 * x0_ref[...]


def _tc_b2(qs, x0, kvs, ksmat, vsum, hist):
    row_spec = pl.BlockSpec((BLK, D), lambda i: (i, 0))
    full = lambda shp: pl.BlockSpec(shp, lambda i: tuple(0 for _ in shp))
    hist_spec = pl.BlockSpec((2, BLK, D), lambda i: (0, i, 0))
    return pl.pallas_call(
        _b2_body,
        grid=(NBLK,),
        in_specs=[row_spec, row_spec, full((D, D)), full((D, D)), full((8, D)),
                  hist_spec],
        out_specs=[row_spec, row_spec, row_spec],
        out_shape=[jax.ShapeDtypeStruct((NP, D), jnp.float32),
                   jax.ShapeDtypeStruct((NP, D), jnp.float32),
                   jax.ShapeDtypeStruct((NP, D), jnp.float32)],
    )(qs, x0, kvs, ksmat, vsum, hist)


def _comb_body(x_ref, acc_ref, a_ref, d2_ref, xn_ref, un_ref):
    acc = acc_ref[...]
    d2 = d2_ref[...]
    gcn = d2 * (acc[0] + acc[1])
    xn = x_ref[...] + BETA * gcn + (1.0 - BETA) * a_ref[...]
    xn_ref[...] = xn
    un_ref[...] = d2 * xn


def _tc_combine(x, acc, a, d2):
    row_spec = pl.BlockSpec((BLK, D), lambda i: (i, 0))
    acc_spec = pl.BlockSpec((2, BLK, D), lambda i: (0, i, 0))
    return pl.pallas_call(
        _comb_body,
        grid=(NBLK,),
        in_specs=[row_spec, acc_spec, row_spec, row_spec],
        out_specs=[row_spec, row_spec],
        out_shape=[jax.ShapeDtypeStruct((NP, D), jnp.float32),
                   jax.ShapeDtypeStruct((NP, D), jnp.float32)],
    )(x, acc, a, d2)


FBLK = 400
NFBLK = N // FBLK


def _final_body(x_ref, acc_ref, a_ref, d2_ref, res_ref,
                wow_ref, wob_ref, bn1g_ref, bn1b_ref, fc1w_ref, fc1b_ref,
                out_ref):
    acc = acc_ref[...]
    gcn = d2_ref[...] * (acc[0] + acc[1])
    x3 = x_ref[...] + BETA * gcn + (1.0 - BETA) * a_ref[...]
    h2 = jnp.dot(x3, wow_ref[...], preferred_element_type=jnp.float32) + wob_ref[...]
    h2 = h2 * bn1g_ref[...] / BN_DIV + bn1b_ref[...]
    h2 = jnp.maximum(h2 + res_ref[...], 0.0)
    out_ref[...] = jnp.dot(h2, fc1w_ref[...], preferred_element_type=jnp.float32) \
        + fc1b_ref[...]


def _tc_final(x2, acc, a, d2, res, wo_w, wo_b, bn1_g, bn1_b, fc1_w, fc1_b):
    row_spec = pl.BlockSpec((FBLK, D), lambda i: (i, 0))
    acc_spec = pl.BlockSpec((2, FBLK, D), lambda i: (0, i, 0))
    full = lambda shp: pl.BlockSpec(shp, lambda i: tuple(0 for _ in shp))
    vec = pl.BlockSpec((D,), lambda i: (0,))
    return pl.pallas_call(
        _final_body,
        grid=(NFBLK,),
        in_specs=[row_spec, acc_spec, row_spec, row_spec, row_spec,
                  full((D, D)), vec, vec, vec, full((D, D)), vec],
        out_specs=pl.BlockSpec((FBLK, D), lambda i: (i, 0)),
        out_shape=jax.ShapeDtypeStruct((N, D), jnp.float32),
    )(x2, acc, a, d2, res, wo_w, wo_b, bn1_g, bn1_b, fc1_w, fc1_b)


def kernel(x, edge_index, fc0_w, fc0_b, bn0_g, bn0_b, wq_w, wq_b, wk_w, wk_b,
           wo_w, wo_b, bn1_g, bn1_b, fc1_w, fc1_b):
    E = edge_index.shape[1]
    pad = jnp.full((EP - E,), PAD_IDX, jnp.int32)
    row_f = jnp.concatenate([edge_index[0], pad])
    col_f = jnp.concatenate([edge_index[1], pad])
    row_p = row_f.reshape(EP // CHUNK, CHUNK)
    col_p = col_f.reshape(EP // CHUNK, CHUNK)
    col_h = col_p
    x_p = jnp.zeros((NP, D), jnp.float32).at[:N].set(x)

    hist = _sc_hist(col_h)
    x0, qs, ks, kvs, vsum = _tc_b1(x_p, fc0_w, fc0_b, bn0_g, bn0_b,
                                   wq_w, wq_b, wk_w, wk_b)
    ks_sum = jnp.sum(ks, axis=0)
    ksmat = jnp.zeros((D, D), jnp.float32).at[:, 0].set(ks_sum)
    a, u0, d2 = _tc_b2(qs, x0, kvs, ksmat, vsum, hist)

    xt, ut = x0, u0
    for _ in range(2):
        acc = _sc_scatter(ut, row_p, col_p)
        xt, ut = _tc_combine(xt, acc, a, d2)
    acc = _sc_scatter(ut, row_p, col_p)
    return _tc_final(xt, acc, a, d2, x0, wo_w, wo_b, bn1_g, bn1_b,
                     fc1_w, fc1_b)


# speedup vs baseline: 5.5711x; 1.0005x over previous
import functools

import numpy as np

import jax
import jax.numpy as jnp
from jax import lax
from jax.experimental import pallas as pl
from jax.experimental.pallas import tpu as pltpu
from jax.experimental.pallas import tpu_sc as plsc

N = 10000
D = 128
NP = 10240
BETA = 0.5
BN_DIV = float(np.sqrt(np.float32(1.0 + 1e-5)))

NW = 32
CHUNK = 128
EC = 80
EPW = EC * CHUNK
EP = NW * EPW
GRP = 8
NGRP = EC // GRP
PAD_IDX = N

ROWS_PER_TILE = NP // 16


def _sc_mesh():
    return plsc.VectorSubcoreMesh(core_axis_name="c", subcore_axis_name="s")


def _hist_body(col_hbm, out_hbm, hist_sh, colv, onesv):
    c = lax.axis_index("c")
    s = lax.axis_index("s")
    wid = s * 2 + c

    def _fill(val, i, _):
        r = i // 8
        q = i % 8
        onesv[r, pl.ds(q * 16, 16)] = jnp.full((16,), val, jnp.float32)
        return 0

    lax.fori_loop(0, CHUNK * 8, functools.partial(_fill, 0.0), 0)
    for k in range(ROWS_PER_TILE // CHUNK):
        pltpu.sync_copy(onesv,
                        hist_sh.at[pl.ds(s * ROWS_PER_TILE + k * CHUNK, CHUNK)])
    lax.fori_loop(0, CHUNK * 8, functools.partial(_fill, 1.0), 0)
    plsc.subcore_barrier()

    pltpu.sync_copy(col_hbm.at[pl.ds(wid * EC, EC)], colv)

    def _scat(j, _):
        pltpu.sync_copy(onesv, hist_sh.at[colv.at[j]], add=True)
        return 0
    lax.fori_loop(0, EC, _scat, 0)

    plsc.subcore_barrier()
    pltpu.sync_copy(hist_sh.at[pl.ds(s * ROWS_PER_TILE, ROWS_PER_TILE)],
                    out_hbm.at[c].at[pl.ds(s * ROWS_PER_TILE, ROWS_PER_TILE)])


def _sc_hist(col_p):
    k = pl.kernel(
        _hist_body,
        out_type=jax.ShapeDtypeStruct((2, NP, D), jnp.float32),
        mesh=_sc_mesh(),
        scratch_types=[
            pltpu.VMEM_SHARED((NP, D), jnp.float32),
            pltpu.VMEM((EC, CHUNK), jnp.int32),
            pltpu.VMEM((CHUNK, D), jnp.float32),
        ],
    )
    return k(col_p)


def _scatter_body(u_hbm, row_hbm, col_hbm, acc_hbm,
                  acc_sh, rowv, colv, rbuf0, rbuf1, sem0, sem1, ssem0, ssem1):
    c = lax.axis_index("c")
    s = lax.axis_index("s")
    wid = s * 2 + c

    def _zfill(i, _):
        r = i // 8
        q = i % 8
        rbuf0[r, pl.ds(q * 16, 16)] = jnp.zeros((16,), jnp.float32)
        return 0
    lax.fori_loop(0, CHUNK * 8, _zfill, 0)

    for k in range(ROWS_PER_TILE // CHUNK):
        pltpu.sync_copy(rbuf0,
                        acc_sh.at[pl.ds(s * ROWS_PER_TILE + k * CHUNK, CHUNK)])
    plsc.subcore_barrier()

    def _group(g, _):
        base = wid * EC + g * GRP
        pltpu.sync_copy(row_hbm.at[pl.ds(base, GRP)], rowv)
        pltpu.sync_copy(col_hbm.at[pl.ds(base, GRP)], colv)
        pltpu.async_copy(u_hbm.at[rowv.at[0]], rbuf0, sem0)

        def _step(j, _):
            @pl.when(j % 2 == 0)
            def _():
                @pl.when(j + 1 < GRP)
                def _():
                    @pl.when(j >= 1)
                    def _():
                        pltpu.make_async_copy(
                            rbuf1, acc_sh.at[colv.at[0]], ssem1).wait()
                    pltpu.async_copy(u_hbm.at[rowv.at[j + 1]], rbuf1, sem1)
                pltpu.make_async_copy(u_hbm.at[rowv.at[j]], rbuf0, sem0).wait()
                pltpu.async_copy(rbuf0, acc_sh.at[colv.at[j]], ssem0, add=True)

            @pl.when(j % 2 == 1)
            def _():
                @pl.when(j + 1 < GRP)
                def _():
                    pltpu.make_async_copy(
                        rbuf0, acc_sh.at[colv.at[0]], ssem0).wait()
                    pltpu.async_copy(u_hbm.at[rowv.at[j + 1]], rbuf0, sem0)
                pltpu.make_async_copy(u_hbm.at[rowv.at[j]], rbuf1, sem1).wait()
                pltpu.async_copy(rbuf1, acc_sh.at[colv.at[j]], ssem1, add=True)
            return 0
        lax.fori_loop(0, GRP, _step, 0)
        pltpu.make_async_copy(rbuf0, acc_sh.at[colv.at[0]], ssem0).wait()
        pltpu.make_async_copy(rbuf1, acc_sh.at[colv.at[0]], ssem1).wait()
        return 0
    lax.fori_loop(0, NGRP, _group, 0)

    plsc.subcore_barrier()
    pltpu.sync_copy(acc_sh.at[pl.ds(s * ROWS_PER_TILE, ROWS_PER_TILE)],
                    acc_hbm.at[c].at[pl.ds(s * ROWS_PER_TILE, ROWS_PER_TILE)])


def _sc_scatter(u, row_p, col_p):
    k = pl.kernel(
        _scatter_body,
        out_type=jax.ShapeDtypeStruct((2, NP, D), jnp.float32),
        mesh=_sc_mesh(),
        scratch_types=[
            pltpu.VMEM_SHARED((NP, D), jnp.float32),
            pltpu.VMEM((GRP, CHUNK), jnp.int32),
            pltpu.VMEM((GRP, CHUNK), jnp.int32),
            pltpu.VMEM((CHUNK, D), jnp.float32),
            pltpu.VMEM((CHUNK, D), jnp.float32),
            pltpu.SemaphoreType.DMA,
            pltpu.SemaphoreType.DMA,
            pltpu.SemaphoreType.DMA,
            pltpu.SemaphoreType.DMA,
        ],
    )
    return k(u, row_p, col_p)


BLK = 512
NBLK = NP // BLK


def _b1_body(x_ref, fc0w_ref, fc0b_ref, bn0g_ref, bn0b_ref,
             wqw_ref, wqb_ref, wkw_ref, wkb_ref,
             x0_ref, qs_ref, ks_ref, kvs_ref, vsum_ref):
    i = pl.program_id(0)

    @pl.when(i == 0)
    def _():
        kvs_ref[...] = jnp.zeros_like(kvs_ref)
        vsum_ref[...] = jnp.zeros_like(vsum_ref)

    xb = x_ref[...]
    h = jnp.dot(xb, fc0w_ref[...], preferred_element_type=jnp.float32)
    h = (h + fc0b_ref[...]) * bn0g_ref[...] / BN_DIV + bn0b_ref[...]
    h = jnp.maximum(h, 0.0)
    rows = i * BLK + lax.broadcasted_iota(jnp.int32, (BLK, 1), 0)
    mask = rows < N
    h = jnp.where(mask, h, 0.0)

    q = jnp.dot(h, wqw_ref[...], preferred_element_type=jnp.float32) + wqb_ref[...]
    kk = jnp.dot(h, wkw_ref[...], preferred_element_type=jnp.float32) + wkb_ref[...]
    qn = jnp.sqrt(jnp.sum(q * q, axis=1, keepdims=True))
    kn = jnp.sqrt(jnp.sum(kk * kk, axis=1, keepdims=True))
    qs = jnp.where(mask, q / jnp.broadcast_to(qn, (BLK, D)), 0.0)
    ks = jnp.where(mask, kk / jnp.broadcast_to(kn, (BLK, D)), 0.0)

    x0_ref[...] = h
    qs_ref[...] = qs
    ks_ref[...] = ks
    kvs_ref[...] += lax.dot_general(ks, h, (((0,), (0,)), ((), ())),
                                    preferred_element_type=jnp.float32)
    vsum_ref[...] += jnp.broadcast_to(jnp.sum(h, axis=0, keepdims=True), (8, D))


def _tc_b1(x_p, fc0_w, fc0_b, bn0_g, bn0_b, wq_w, wq_b, wk_w, wk_b):
    row_spec = pl.BlockSpec((BLK, D), lambda i: (i, 0))
    full = lambda shp: pl.BlockSpec(shp, lambda i: tuple(0 for _ in shp))
    vec = pl.BlockSpec((D,), lambda i: (0,))
    return pl.pallas_call(
        _b1_body,
        grid=(NBLK,),
        in_specs=[row_spec, full((D, D)), vec, vec, vec,
                  full((D, D)), vec, full((D, D)), vec],
        out_specs=[row_spec, row_spec, row_spec, full((D, D)), full((8, D))],
        out_shape=[jax.ShapeDtypeStruct((NP, D), jnp.float32),
                   jax.ShapeDtypeStruct((NP, D), jnp.float32),
                   jax.ShapeDtypeStruct((NP, D), jnp.float32),
                   jax.ShapeDtypeStruct((D, D), jnp.float32),
                   jax.ShapeDtypeStruct((8, D), jnp.float32)],
    )(x_p, fc0_w, fc0_b, bn0_g, bn0_b, wq_w, wq_b, wk_w, wk_b)


def _b2_body(qs_ref, x0_ref, kvs_ref, ksmat_ref, vsum_ref, hist_ref,
             a_ref, u0_ref, d2_ref):
    qs = qs_ref[...]
    hb = hist_ref[...]
    deg = hb[0, :, 0:1] + hb[1, :, 0:1]
    dis = jnp.where(deg > 0, lax.rsqrt(jnp.maximum(deg, 1e-30)), 0.0)
    d2 = jnp.broadcast_to(dis, (BLK, D))

    num = jnp.dot(qs, kvs_ref[...], preferred_element_type=jnp.float32)
    num = num + jnp.broadcast_to(vsum_ref[0:1, :], (BLK, D))
    den = jnp.dot(qs, ksmat_ref[...], preferred_element_type=jnp.float32)
    den = den[:, 0:1] + 1.0
    a_ref[...] = num / jnp.broadcast_to(den, (BLK, D))
    d2_ref[...] = d2
    u0_ref[...] = d2 * x0_ref[...]


def _tc_b2(qs, x0, kvs, ksmat, vsum, hist):
    row_spec = pl.BlockSpec((BLK, D), lambda i: (i, 0))
    full = lambda shp: pl.BlockSpec(shp, lambda i: tuple(0 for _ in shp))
    hist_spec = pl.BlockSpec((2, BLK, D), lambda i: (0, i, 0))
    return pl.pallas_call(
        _b2_body,
        grid=(NBLK,),
        in_specs=[row_spec, row_spec, full((D, D)), full((D, D)), full((8, D)),
                  hist_spec],
        out_specs=[row_spec, row_spec, row_spec],
        out_shape=[jax.ShapeDtypeStruct((NP, D), jnp.float32),
                   jax.ShapeDtypeStruct((NP, D), jnp.float32),
                   jax.ShapeDtypeStruct((NP, D), jnp.float32)],
    )(qs, x0, kvs, ksmat, vsum, hist)


def _comb_body(x_ref, acc_ref, a_ref, d2_ref, xn_ref, un_ref):
    acc = acc_ref[...]
    d2 = d2_ref[...]
    gcn = d2 * (acc[0] + acc[1])
    xn = x_ref[...] + BETA * gcn + (1.0 - BETA) * a_ref[...]
    xn_ref[...] = xn
    un_ref[...] = d2 * xn


def _tc_combine(x, acc, a, d2):
    row_spec = pl.BlockSpec((BLK, D), lambda i: (i, 0))
    acc_spec = pl.BlockSpec((2, BLK, D), lambda i: (0, i, 0))
    return pl.pallas_call(
        _comb_body,
        grid=(NBLK,),
        in_specs=[row_spec, acc_spec, row_spec, row_spec],
        out_specs=[row_spec, row_spec],
        out_shape=[jax.ShapeDtypeStruct((NP, D), jnp.float32),
                   jax.ShapeDtypeStruct((NP, D), jnp.float32)],
    )(x, acc, a, d2)


FBLK = 400
NFBLK = N // FBLK


def _final_body(x_ref, acc_ref, a_ref, d2_ref, res_ref,
                wow_ref, wob_ref, bn1g_ref, bn1b_ref, fc1w_ref, fc1b_ref,
                out_ref):
    acc = acc_ref[...]
    gcn = d2_ref[...] * (acc[0] + acc[1])
    x3 = x_ref[...] + BETA * gcn + (1.0 - BETA) * a_ref[...]
    h2 = jnp.dot(x3, wow_ref[...], preferred_element_type=jnp.float32) + wob_ref[...]
    h2 = h2 * bn1g_ref[...] / BN_DIV + bn1b_ref[...]
    h2 = jnp.maximum(h2 + res_ref[...], 0.0)
    out_ref[...] = jnp.dot(h2, fc1w_ref[...], preferred_element_type=jnp.float32) \
        + fc1b_ref[...]


def _tc_final(x2, acc, a, d2, res, wo_w, wo_b, bn1_g, bn1_b, fc1_w, fc1_b):
    row_spec = pl.BlockSpec((FBLK, D), lambda i: (i, 0))
    acc_spec = pl.BlockSpec((2, FBLK, D), lambda i: (0, i, 0))
    full = lambda shp: pl.BlockSpec(shp, lambda i: tuple(0 for _ in shp))
    vec = pl.BlockSpec((D,), lambda i: (0,))
    return pl.pallas_call(
        _final_body,
        grid=(NFBLK,),
        in_specs=[row_spec, acc_spec, row_spec, row_spec, row_spec,
                  full((D, D)), vec, vec, vec, full((D, D)), vec],
        out_specs=pl.BlockSpec((FBLK, D), lambda i: (i, 0)),
        out_shape=jax.ShapeDtypeStruct((N, D), jnp.float32),
    )(x2, acc, a, d2, res, wo_w, wo_b, bn1_g, bn1_b, fc1_w, fc1_b)


def kernel(x, edge_index, fc0_w, fc0_b, bn0_g, bn0_b, wq_w, wq_b, wk_w, wk_b,
           wo_w, wo_b, bn1_g, bn1_b, fc1_w, fc1_b):
    E = edge_index.shape[1]
    pad = jnp.full((EP - E,), PAD_IDX, jnp.int32)
    row_f = jnp.concatenate([edge_index[0], pad])
    col_f = jnp.concatenate([edge_index[1], pad])
    row_p = row_f.reshape(EP // CHUNK, CHUNK)
    col_p = col_f.reshape(EP // CHUNK, CHUNK)
    col_h = col_p
    x_p = jnp.zeros((NP, D), jnp.float32).at[:N].set(x)

    hist = _sc_hist(col_h)
    x0, qs, ks, kvs, vsum = _tc_b1(x_p, fc0_w, fc0_b, bn0_g, bn0_b,
                                   wq_w, wq_b, wk_w, wk_b)
    ks_sum = jnp.sum(ks, axis=0)
    ksmat = jnp.zeros((D, D), jnp.float32).at[:, 0].set(ks_sum)
    a, u0, d2 = _tc_b2(qs, x0, kvs, ksmat, vsum, hist)

    xt, ut = x0, u0
    for _ in range(2):
        acc = _sc_scatter(ut, row_p, col_p)
        xt, ut = _tc_combine(xt, acc, a, d2)
    acc = _sc_scatter(ut, row_p, col_p)
    return _tc_final(xt, acc, a, d2, x0, wo_w, wo_b, bn1_g, bn1_b,
                     fc1_w, fc1_b)


# speedup vs baseline: 5.9174x; 1.0621x over previous
import functools

import numpy as np

import jax
import jax.numpy as jnp
from jax import lax
from jax.experimental import pallas as pl
from jax.experimental.pallas import tpu as pltpu
from jax.experimental.pallas import tpu_sc as plsc

N = 10000
D = 128
NP = 10240
BETA = 0.5
BN_DIV = float(np.sqrt(np.float32(1.0 + 1e-5)))

NW = 32
CHUNK = 128
EC = 80
EPW = EC * CHUNK
EP = NW * EPW
GRP = 8
NGRP = EC // GRP
SCH = 64
SGRP = 16
SEC = EPW // SCH
SNGRP = SEC // SGRP
PAD_IDX = N

ROWS_PER_TILE = NP // 16


def _sc_mesh():
    return plsc.VectorSubcoreMesh(core_axis_name="c", subcore_axis_name="s")


def _hist_body(col_hbm, out_hbm, hist_sh, colv, onesv):
    c = lax.axis_index("c")
    s = lax.axis_index("s")
    wid = s * 2 + c

    def _fill(val, i, _):
        r = i // 8
        q = i % 8
        onesv[r, pl.ds(q * 16, 16)] = jnp.full((16,), val, jnp.float32)
        return 0

    lax.fori_loop(0, CHUNK * 8, functools.partial(_fill, 0.0), 0)
    for k in range(ROWS_PER_TILE // CHUNK):
        pltpu.sync_copy(onesv,
                        hist_sh.at[pl.ds(s * ROWS_PER_TILE + k * CHUNK, CHUNK)])
    lax.fori_loop(0, CHUNK * 8, functools.partial(_fill, 1.0), 0)
    plsc.subcore_barrier()

    pltpu.sync_copy(col_hbm.at[pl.ds(wid * EC, EC)], colv)

    def _scat(j, _):
        pltpu.sync_copy(onesv, hist_sh.at[colv.at[j]], add=True)
        return 0
    lax.fori_loop(0, EC, _scat, 0)

    plsc.subcore_barrier()
    pltpu.sync_copy(hist_sh.at[pl.ds(s * ROWS_PER_TILE, ROWS_PER_TILE)],
                    out_hbm.at[c].at[pl.ds(s * ROWS_PER_TILE, ROWS_PER_TILE)])


def _sc_hist(col_p):
    k = pl.kernel(
        _hist_body,
        out_type=jax.ShapeDtypeStruct((2, NP, D), jnp.float32),
        mesh=_sc_mesh(),
        scratch_types=[
            pltpu.VMEM_SHARED((NP, D), jnp.float32),
            pltpu.VMEM((EC, CHUNK), jnp.int32),
            pltpu.VMEM((CHUNK, D), jnp.float32),
        ],
    )
    return k(col_p)


def _scatter_body(u_hbm, row_hbm, col_hbm, acc_hbm,
                  acc_sh, rowv, colv, rb0, rb1, rb2, rb3,
                  gs0, gs1, gs2, gs3):
    c = lax.axis_index("c")
    s = lax.axis_index("s")
    wid = s * 2 + c
    rbs = (rb0, rb1, rb2, rb3)
    gss = (gs0, gs1, gs2, gs3)

    def _zfill(i, _):
        r = i // 8
        q = i % 8
        rb0[r, pl.ds(q * 16, 16)] = jnp.zeros((16,), jnp.float32)
        return 0
    lax.fori_loop(0, SCH * 8, _zfill, 0)

    for k in range(ROWS_PER_TILE // SCH):
        pltpu.sync_copy(rb0,
                        acc_sh.at[pl.ds(s * ROWS_PER_TILE + k * SCH, SCH)])
    plsc.subcore_barrier()

    def _group(g, _):
        base = wid * SEC + g * SGRP
        pltpu.sync_copy(row_hbm.at[pl.ds(base, SGRP)], rowv)
        pltpu.sync_copy(col_hbm.at[pl.ds(base, SGRP)], colv)
        for b in range(3):
            pltpu.async_copy(u_hbm.at[rowv.at[b]], rbs[b], gss[b])

        def _step(j, _):
            for b in range(4):
                @pl.when(j % 4 == b)
                def _(b=b):
                    nb = (b + 3) % 4
                    @pl.when(j + 3 < SGRP)
                    def _():
                        pltpu.async_copy(u_hbm.at[rowv.at[j + 3]],
                                         rbs[nb], gss[nb])
                    pltpu.make_async_copy(u_hbm.at[rowv.at[j]],
                                          rbs[b], gss[b]).wait()
                    pltpu.sync_copy(rbs[b], acc_sh.at[colv.at[j]], add=True)
            return 0
        lax.fori_loop(0, SGRP, _step, 0)
        return 0
    lax.fori_loop(0, SNGRP, _group, 0)

    plsc.subcore_barrier()
    pltpu.sync_copy(acc_sh.at[pl.ds(s * ROWS_PER_TILE, ROWS_PER_TILE)],
                    acc_hbm.at[c].at[pl.ds(s * ROWS_PER_TILE, ROWS_PER_TILE)])


def _sc_scatter(u, row_p, col_p):
    k = pl.kernel(
        _scatter_body,
        out_type=jax.ShapeDtypeStruct((2, NP, D), jnp.float32),
        mesh=_sc_mesh(),
        scratch_types=[
            pltpu.VMEM_SHARED((NP, D), jnp.float32),
            pltpu.VMEM((SGRP, SCH), jnp.int32),
            pltpu.VMEM((SGRP, SCH), jnp.int32),
            pltpu.VMEM((SCH, D), jnp.float32),
            pltpu.VMEM((SCH, D), jnp.float32),
            pltpu.VMEM((SCH, D), jnp.float32),
            pltpu.VMEM((SCH, D), jnp.float32),
            pltpu.SemaphoreType.DMA,
            pltpu.SemaphoreType.DMA,
            pltpu.SemaphoreType.DMA,
            pltpu.SemaphoreType.DMA,
        ],
    )
    return k(u, row_p, col_p)


BLK = 512
NBLK = NP // BLK


def _b1_body(x_ref, fc0w_ref, fc0b_ref, bn0g_ref, bn0b_ref,
             wqw_ref, wqb_ref, wkw_ref, wkb_ref,
             x0_ref, qs_ref, ks_ref, kvs_ref, vsum_ref):
    i = pl.program_id(0)

    @pl.when(i == 0)
    def _():
        kvs_ref[...] = jnp.zeros_like(kvs_ref)
        vsum_ref[...] = jnp.zeros_like(vsum_ref)

    xb = x_ref[...]
    h = jnp.dot(xb, fc0w_ref[...], preferred_element_type=jnp.float32)
    h = (h + fc0b_ref[...]) * bn0g_ref[...] / BN_DIV + bn0b_ref[...]
    h = jnp.maximum(h, 0.0)
    rows = i * BLK + lax.broadcasted_iota(jnp.int32, (BLK, 1), 0)
    mask = rows < N
    h = jnp.where(mask, h, 0.0)

    q = jnp.dot(h, wqw_ref[...], preferred_element_type=jnp.float32) + wqb_ref[...]
    kk = jnp.dot(h, wkw_ref[...], preferred_element_type=jnp.float32) + wkb_ref[...]
    qn = jnp.sqrt(jnp.sum(q * q, axis=1, keepdims=True))
    kn = jnp.sqrt(jnp.sum(kk * kk, axis=1, keepdims=True))
    qs = jnp.where(mask, q / jnp.broadcast_to(qn, (BLK, D)), 0.0)
    ks = jnp.where(mask, kk / jnp.broadcast_to(kn, (BLK, D)), 0.0)

    x0_ref[...] = h
    qs_ref[...] = qs
    ks_ref[...] = ks
    kvs_ref[...] += lax.dot_general(ks, h, (((0,), (0,)), ((), ())),
                                    preferred_element_type=jnp.float32)
    vsum_ref[...] += jnp.broadcast_to(jnp.sum(h, axis=0, keepdims=True), (8, D))


def _tc_b1(x_p, fc0_w, fc0_b, bn0_g, bn0_b, wq_w, wq_b, wk_w, wk_b):
    row_spec = pl.BlockSpec((BLK, D), lambda i: (i, 0))
    full = lambda shp: pl.BlockSpec(shp, lambda i: tuple(0 for _ in shp))
    vec = pl.BlockSpec((D,), lambda i: (0,))
    return pl.pallas_call(
        _b1_body,
        grid=(NBLK,),
        in_specs=[row_spec, full((D, D)), vec, vec, vec,
                  full((D, D)), vec, full((D, D)), vec],
        out_specs=[row_spec, row_spec, row_spec, full((D, D)), full((8, D))],
        out_shape=[jax.ShapeDtypeStruct((NP, D), jnp.float32),
                   jax.ShapeDtypeStruct((NP, D), jnp.float32),
                   jax.ShapeDtypeStruct((NP, D), jnp.float32),
                   jax.ShapeDtypeStruct((D, D), jnp.float32),
                   jax.ShapeDtypeStruct((8, D), jnp.float32)],
    )(x_p, fc0_w, fc0_b, bn0_g, bn0_b, wq_w, wq_b, wk_w, wk_b)


def _b2_body(qs_ref, x0_ref, kvs_ref, ksmat_ref, vsum_ref, hist_ref,
             a_ref, u0_ref, d2_ref):
    qs = qs_ref[...]
    hb = hist_ref[...]
    deg = hb[0, :, 0:1] + hb[1, :, 0:1]
    dis = jnp.where(deg > 0, lax.rsqrt(jnp.maximum(deg, 1e-30)), 0.0)
    d2 = jnp.broadcast_to(dis, (BLK, D))

    num = jnp.dot(qs, kvs_ref[...], preferred_element_type=jnp.float32)
    num = num + jnp.broadcast_to(vsum_ref[0:1, :], (BLK, D))
    den = jnp.dot(qs, ksmat_ref[...], preferred_element_type=jnp.float32)
    den = den[:, 0:1] + 1.0
    a_ref[...] = num / jnp.broadcast_to(den, (BLK, D))
    d2_ref[...] = d2
    u0_ref[...] = d2 * x0_ref[...]


def _tc_b2(qs, x0, kvs, ksmat, vsum, hist):
    row_spec = pl.BlockSpec((BLK, D), lambda i: (i, 0))
    full = lambda shp: pl.BlockSpec(shp, lambda i: tuple(0 for _ in shp))
    hist_spec = pl.BlockSpec((2, BLK, D), lambda i: (0, i, 0))
    return pl.pallas_call(
        _b2_body,
        grid=(NBLK,),
        in_specs=[row_spec, row_spec, full((D, D)), full((D, D)), full((8, D)),
                  hist_spec],
        out_specs=[row_spec, row_spec, row_spec],
        out_shape=[jax.ShapeDtypeStruct((NP, D), jnp.float32),
                   jax.ShapeDtypeStruct((NP, D), jnp.float32),
                   jax.ShapeDtypeStruct((NP, D), jnp.float32)],
    )(qs, x0, kvs, ksmat, vsum, hist)


def _comb_body(x_ref, acc_ref, a_ref, d2_ref, xn_ref, un_ref):
    acc = acc_ref[...]
    d2 = d2_ref[...]
    gcn = d2 * (acc[0] + acc[1])
    xn = x_ref[...] + BETA * gcn + (1.0 - BETA) * a_ref[...]
    xn_ref[...] = xn
    un_ref[...] = d2 * xn


def _tc_combine(x, acc, a, d2):
    row_spec = pl.BlockSpec((BLK, D), lambda i: (i, 0))
    acc_spec = pl.BlockSpec((2, BLK, D), lambda i: (0, i, 0))
    return pl.pallas_call(
        _comb_body,
        grid=(NBLK,),
        in_specs=[row_spec, acc_spec, row_spec, row_spec],
        out_specs=[row_spec, row_spec],
        out_shape=[jax.ShapeDtypeStruct((NP, D), jnp.float32),
                   jax.ShapeDtypeStruct((NP, D), jnp.float32)],
    )(x, acc, a, d2)


FBLK = 400
NFBLK = N // FBLK


def _final_body(x_ref, acc_ref, a_ref, d2_ref, res_ref,
                wow_ref, wob_ref, bn1g_ref, bn1b_ref, fc1w_ref, fc1b_ref,
                out_ref):
    acc = acc_ref[...]
    gcn = d2_ref[...] * (acc[0] + acc[1])
    x3 = x_ref[...] + BETA * gcn + (1.0 - BETA) * a_ref[...]
    h2 = jnp.dot(x3, wow_ref[...], preferred_element_type=jnp.float32) + wob_ref[...]
    h2 = h2 * bn1g_ref[...] / BN_DIV + bn1b_ref[...]
    h2 = jnp.maximum(h2 + res_ref[...], 0.0)
    out_ref[...] = jnp.dot(h2, fc1w_ref[...], preferred_element_type=jnp.float32) \
        + fc1b_ref[...]


def _tc_final(x2, acc, a, d2, res, wo_w, wo_b, bn1_g, bn1_b, fc1_w, fc1_b):
    row_spec = pl.BlockSpec((FBLK, D), lambda i: (i, 0))
    acc_spec = pl.BlockSpec((2, FBLK, D), lambda i: (0, i, 0))
    full = lambda shp: pl.BlockSpec(shp, lambda i: tuple(0 for _ in shp))
    vec = pl.BlockSpec((D,), lambda i: (0,))
    return pl.pallas_call(
        _final_body,
        grid=(NFBLK,),
        in_specs=[row_spec, acc_spec, row_spec, row_spec, row_spec,
                  full((D, D)), vec, vec, vec, full((D, D)), vec],
        out_specs=pl.BlockSpec((FBLK, D), lambda i: (i, 0)),
        out_shape=jax.ShapeDtypeStruct((N, D), jnp.float32),
    )(x2, acc, a, d2, res, wo_w, wo_b, bn1_g, bn1_b, fc1_w, fc1_b)


def kernel(x, edge_index, fc0_w, fc0_b, bn0_g, bn0_b, wq_w, wq_b, wk_w, wk_b,
           wo_w, wo_b, bn1_g, bn1_b, fc1_w, fc1_b):
    E = edge_index.shape[1]
    pad = jnp.full((EP - E,), PAD_IDX, jnp.int32)
    row_f = jnp.concatenate([edge_index[0], pad])
    col_f = jnp.concatenate([edge_index[1], pad])
    row_p = row_f.reshape(EP // SCH, SCH)
    col_p = col_f.reshape(EP // SCH, SCH)
    col_h = col_f.reshape(EP // CHUNK, CHUNK)
    x_p = jnp.zeros((NP, D), jnp.float32).at[:N].set(x)

    hist = _sc_hist(col_h)
    x0, qs, ks, kvs, vsum = _tc_b1(x_p, fc0_w, fc0_b, bn0_g, bn0_b,
                                   wq_w, wq_b, wk_w, wk_b)
    ks_sum = jnp.sum(ks, axis=0)
    ksmat = jnp.zeros((D, D), jnp.float32).at[:, 0].set(ks_sum)
    a, u0, d2 = _tc_b2(qs, x0, kvs, ksmat, vsum, hist)

    xt, ut = x0, u0
    for _ in range(2):
        acc = _sc_scatter(ut, row_p, col_p)
        xt, ut = _tc_combine(xt, acc, a, d2)
    acc = _sc_scatter(ut, row_p, col_p)
    return _tc_final(xt, acc, a, d2, x0, wo_w, wo_b, bn1_g, bn1_b,
                     fc1_w, fc1_b)


# speedup vs baseline: 5.9456x; 1.0048x over previous
import functools

import numpy as np

import jax
import jax.numpy as jnp
from jax import lax
from jax.experimental import pallas as pl
from jax.experimental.pallas import tpu as pltpu
from jax.experimental.pallas import tpu_sc as plsc

N = 10000
D = 128
NP = 10240
BETA = 0.5
BN_DIV = float(np.sqrt(np.float32(1.0 + 1e-5)))

NW = 32
CHUNK = 128
EC = 80
EPW = EC * CHUNK
EP = NW * EPW
GRP = 8
NGRP = EC // GRP
SCH = 64
SGRP = 16
SEC = EPW // SCH
SNGRP = SEC // SGRP
PAD_IDX = N

ROWS_PER_TILE = NP // 16


def _sc_mesh():
    return plsc.VectorSubcoreMesh(core_axis_name="c", subcore_axis_name="s")


def _hist_body(col_hbm, out_hbm, hist_sh, colv, onesv):
    c = lax.axis_index("c")
    s = lax.axis_index("s")
    wid = s * 2 + c

    def _fill(val, i, _):
        r = i // 8
        q = i % 8
        onesv[r, pl.ds(q * 16, 16)] = jnp.full((16,), val, jnp.float32)
        return 0

    lax.fori_loop(0, CHUNK * 8, functools.partial(_fill, 0.0), 0)
    for k in range(ROWS_PER_TILE // CHUNK):
        pltpu.sync_copy(onesv,
                        hist_sh.at[pl.ds(s * ROWS_PER_TILE + k * CHUNK, CHUNK)])
    lax.fori_loop(0, CHUNK * 8, functools.partial(_fill, 1.0), 0)
    plsc.subcore_barrier()

    pltpu.sync_copy(col_hbm.at[pl.ds(wid * EC, EC)], colv)

    def _scat(j, _):
        pltpu.sync_copy(onesv, hist_sh.at[colv.at[j]], add=True)
        return 0
    lax.fori_loop(0, EC, _scat, 0)

    plsc.subcore_barrier()
    pltpu.sync_copy(hist_sh.at[pl.ds(s * ROWS_PER_TILE, ROWS_PER_TILE)],
                    out_hbm.at[c].at[pl.ds(s * ROWS_PER_TILE, ROWS_PER_TILE)])


def _sc_hist(col_p):
    k = pl.kernel(
        _hist_body,
        out_type=jax.ShapeDtypeStruct((2, NP, D), jnp.float32),
        mesh=_sc_mesh(),
        scratch_types=[
            pltpu.VMEM_SHARED((NP, D), jnp.float32),
            pltpu.VMEM((EC, CHUNK), jnp.int32),
            pltpu.VMEM((CHUNK, D), jnp.float32),
        ],
    )
    return k(col_p)


def _scatter_body(u_hbm, row_hbm, col_hbm, acc_hbm,
                  acc_sh, rowv, colv, rb0, rb1, rb2, rb3, rb4,
                  gs0, gs1, gs2, gs3, gs4):
    c = lax.axis_index("c")
    s = lax.axis_index("s")
    wid = s * 2 + c
    rbs = (rb0, rb1, rb2, rb3, rb4)
    gss = (gs0, gs1, gs2, gs3, gs4)

    def _zfill(i, _):
        r = i // 8
        q = i % 8
        rb0[r, pl.ds(q * 16, 16)] = jnp.zeros((16,), jnp.float32)
        return 0
    lax.fori_loop(0, SCH * 8, _zfill, 0)

    for k in range(ROWS_PER_TILE // SCH):
        pltpu.sync_copy(rb0,
                        acc_sh.at[pl.ds(s * ROWS_PER_TILE + k * SCH, SCH)])
    plsc.subcore_barrier()

    def _group(g, _):
        base = wid * SEC + g * SGRP
        pltpu.sync_copy(row_hbm.at[pl.ds(base, SGRP)], rowv)
        pltpu.sync_copy(col_hbm.at[pl.ds(base, SGRP)], colv)
        for b in range(4):
            pltpu.async_copy(u_hbm.at[rowv.at[b]], rbs[b], gss[b])

        def _step(j, _):
            for b in range(5):
                @pl.when(j % 5 == b)
                def _(b=b):
                    nb = (b + 4) % 5
                    @pl.when(j + 4 < SGRP)
                    def _():
                        pltpu.async_copy(u_hbm.at[rowv.at[j + 4]],
                                         rbs[nb], gss[nb])
                    pltpu.make_async_copy(u_hbm.at[rowv.at[j]],
                                          rbs[b], gss[b]).wait()
                    pltpu.sync_copy(rbs[b], acc_sh.at[colv.at[j]], add=True)
            return 0
        lax.fori_loop(0, SGRP, _step, 0)
        return 0
    lax.fori_loop(0, SNGRP, _group, 0)

    plsc.subcore_barrier()
    pltpu.sync_copy(acc_sh.at[pl.ds(s * ROWS_PER_TILE, ROWS_PER_TILE)],
                    acc_hbm.at[c].at[pl.ds(s * ROWS_PER_TILE, ROWS_PER_TILE)])


def _sc_scatter(u, row_p, col_p):
    k = pl.kernel(
        _scatter_body,
        out_type=jax.ShapeDtypeStruct((2, NP, D), jnp.float32),
        mesh=_sc_mesh(),
        scratch_types=[
            pltpu.VMEM_SHARED((NP, D), jnp.float32),
            pltpu.VMEM((SGRP, SCH), jnp.int32),
            pltpu.VMEM((SGRP, SCH), jnp.int32),
            pltpu.VMEM((SCH, D), jnp.float32),
            pltpu.VMEM((SCH, D), jnp.float32),
            pltpu.VMEM((SCH, D), jnp.float32),
            pltpu.VMEM((SCH, D), jnp.float32),
            pltpu.VMEM((SCH, D), jnp.float32),
            pltpu.SemaphoreType.DMA,
            pltpu.SemaphoreType.DMA,
            pltpu.SemaphoreType.DMA,
            pltpu.SemaphoreType.DMA,
            pltpu.SemaphoreType.DMA,
        ],
    )
    return k(u, row_p, col_p)


BLK = 512
NBLK = NP // BLK


def _b1_body(x_ref, fc0w_ref, fc0b_ref, bn0g_ref, bn0b_ref,
             wqw_ref, wqb_ref, wkw_ref, wkb_ref,
             x0_ref, qs_ref, ks_ref, kvs_ref, vsum_ref):
    i = pl.program_id(0)

    @pl.when(i == 0)
    def _():
        kvs_ref[...] = jnp.zeros_like(kvs_ref)
        vsum_ref[...] = jnp.zeros_like(vsum_ref)

    xb = x_ref[...]
    h = jnp.dot(xb, fc0w_ref[...], preferred_element_type=jnp.float32)
    h = (h + fc0b_ref[...]) * bn0g_ref[...] / BN_DIV + bn0b_ref[...]
    h = jnp.maximum(h, 0.0)
    rows = i * BLK + lax.broadcasted_iota(jnp.int32, (BLK, 1), 0)
    mask = rows < N
    h = jnp.where(mask, h, 0.0)

    q = jnp.dot(h, wqw_ref[...], preferred_element_type=jnp.float32) + wqb_ref[...]
    kk = jnp.dot(h, wkw_ref[...], preferred_element_type=jnp.float32) + wkb_ref[...]
    qn = jnp.sqrt(jnp.sum(q * q, axis=1, keepdims=True))
    kn = jnp.sqrt(jnp.sum(kk * kk, axis=1, keepdims=True))
    qs = jnp.where(mask, q / jnp.broadcast_to(qn, (BLK, D)), 0.0)
    ks = jnp.where(mask, kk / jnp.broadcast_to(kn, (BLK, D)), 0.0)

    x0_ref[...] = h
    qs_ref[...] = qs
    ks_ref[...] = ks
    kvs_ref[...] += lax.dot_general(ks, h, (((0,), (0,)), ((), ())),
                                    preferred_element_type=jnp.float32)
    vsum_ref[...] += jnp.broadcast_to(jnp.sum(h, axis=0, keepdims=True), (8, D))


def _tc_b1(x_p, fc0_w, fc0_b, bn0_g, bn0_b, wq_w, wq_b, wk_w, wk_b):
    row_spec = pl.BlockSpec((BLK, D), lambda i: (i, 0))
    full = lambda shp: pl.BlockSpec(shp, lambda i: tuple(0 for _ in shp))
    vec = pl.BlockSpec((D,), lambda i: (0,))
    return pl.pallas_call(
        _b1_body,
        grid=(NBLK,),
        in_specs=[row_spec, full((D, D)), vec, vec, vec,
                  full((D, D)), vec, full((D, D)), vec],
        out_specs=[row_spec, row_spec, row_spec, full((D, D)), full((8, D))],
        out_shape=[jax.ShapeDtypeStruct((NP, D), jnp.float32),
                   jax.ShapeDtypeStruct((NP, D), jnp.float32),
                   jax.ShapeDtypeStruct((NP, D), jnp.float32),
                   jax.ShapeDtypeStruct((D, D), jnp.float32),
                   jax.ShapeDtypeStruct((8, D), jnp.float32)],
    )(x_p, fc0_w, fc0_b, bn0_g, bn0_b, wq_w, wq_b, wk_w, wk_b)


def _b2_body(qs_ref, x0_ref, kvs_ref, ksmat_ref, vsum_ref, hist_ref,
             a_ref, u0_ref, d2_ref):
    qs = qs_ref[...]
    hb = hist_ref[...]
    deg = hb[0, :, 0:1] + hb[1, :, 0:1]
    dis = jnp.where(deg > 0, lax.rsqrt(jnp.maximum(deg, 1e-30)), 0.0)
    d2 = jnp.broadcast_to(dis, (BLK, D))

    num = jnp.dot(qs, kvs_ref[...], preferred_element_type=jnp.float32)
    num = num + jnp.broadcast_to(vsum_ref[0:1, :], (BLK, D))
    den = jnp.dot(qs, ksmat_ref[...], preferred_element_type=jnp.float32)
    den = den[:, 0:1] + 1.0
    a_ref[...] = num / jnp.broadcast_to(den, (BLK, D))
    d2_ref[...] = d2
    u0_ref[...] = d2 * x0_ref[...]


def _tc_b2(qs, x0, kvs, ksmat, vsum, hist):
    row_spec = pl.BlockSpec((BLK, D), lambda i: (i, 0))
    full = lambda shp: pl.BlockSpec(shp, lambda i: tuple(0 for _ in shp))
    hist_spec = pl.BlockSpec((2, BLK, D), lambda i: (0, i, 0))
    return pl.pallas_call(
        _b2_body,
        grid=(NBLK,),
        in_specs=[row_spec, row_spec, full((D, D)), full((D, D)), full((8, D)),
                  hist_spec],
        out_specs=[row_spec, row_spec, row_spec],
        out_shape=[jax.ShapeDtypeStruct((NP, D), jnp.float32),
                   jax.ShapeDtypeStruct((NP, D), jnp.float32),
                   jax.ShapeDtypeStruct((NP, D), jnp.float32)],
    )(qs, x0, kvs, ksmat, vsum, hist)


def _comb_body(x_ref, acc_ref, a_ref, d2_ref, xn_ref, un_ref):
    acc = acc_ref[...]
    d2 = d2_ref[...]
    gcn = d2 * (acc[0] + acc[1])
    xn = x_ref[...] + BETA * gcn + (1.0 - BETA) * a_ref[...]
    xn_ref[...] = xn
    un_ref[...] = d2 * xn


def _tc_combine(x, acc, a, d2):
    row_spec = pl.BlockSpec((BLK, D), lambda i: (i, 0))
    acc_spec = pl.BlockSpec((2, BLK, D), lambda i: (0, i, 0))
    return pl.pallas_call(
        _comb_body,
        grid=(NBLK,),
        in_specs=[row_spec, acc_spec, row_spec, row_spec],
        out_specs=[row_spec, row_spec],
        out_shape=[jax.ShapeDtypeStruct((NP, D), jnp.float32),
                   jax.ShapeDtypeStruct((NP, D), jnp.float32)],
    )(x, acc, a, d2)


FBLK = 400
NFBLK = N // FBLK


def _final_body(x_ref, acc_ref, a_ref, d2_ref, res_ref,
                wow_ref, wob_ref, bn1g_ref, bn1b_ref, fc1w_ref, fc1b_ref,
                out_ref):
    acc = acc_ref[...]
    gcn = d2_ref[...] * (acc[0] + acc[1])
    x3 = x_ref[...] + BETA * gcn + (1.0 - BETA) * a_ref[...]
    h2 = jnp.dot(x3, wow_ref[...], preferred_element_type=jnp.float32) + wob_ref[...]
    h2 = h2 * bn1g_ref[...] / BN_DIV + bn1b_ref[...]
    h2 = jnp.maximum(h2 + res_ref[...], 0.0)
    out_ref[...] = jnp.dot(h2, fc1w_ref[...], preferred_element_type=jnp.float32) \
        + fc1b_ref[...]


def _tc_final(x2, acc, a, d2, res, wo_w, wo_b, bn1_g, bn1_b, fc1_w, fc1_b):
    row_spec = pl.BlockSpec((FBLK, D), lambda i: (i, 0))
    acc_spec = pl.BlockSpec((2, FBLK, D), lambda i: (0, i, 0))
    full = lambda shp: pl.BlockSpec(shp, lambda i: tuple(0 for _ in shp))
    vec = pl.BlockSpec((D,), lambda i: (0,))
    return pl.pallas_call(
        _final_body,
        grid=(NFBLK,),
        in_specs=[row_spec, acc_spec, row_spec, row_spec, row_spec,
                  full((D, D)), vec, vec, vec, full((D, D)), vec],
        out_specs=pl.BlockSpec((FBLK, D), lambda i: (i, 0)),
        out_shape=jax.ShapeDtypeStruct((N, D), jnp.float32),
    )(x2, acc, a, d2, res, wo_w, wo_b, bn1_g, bn1_b, fc1_w, fc1_b)


def kernel(x, edge_index, fc0_w, fc0_b, bn0_g, bn0_b, wq_w, wq_b, wk_w, wk_b,
           wo_w, wo_b, bn1_g, bn1_b, fc1_w, fc1_b):
    E = edge_index.shape[1]
    pad = jnp.full((EP - E,), PAD_IDX, jnp.int32)
    row_f = jnp.concatenate([edge_index[0], pad])
    col_f = jnp.concatenate([edge_index[1], pad])
    row_p = row_f.reshape(EP // SCH, SCH)
    col_p = col_f.reshape(EP // SCH, SCH)
    col_h = col_f.reshape(EP // CHUNK, CHUNK)
    x_p = jnp.zeros((NP, D), jnp.float32).at[:N].set(x)

    hist = _sc_hist(col_h)
    x0, qs, ks, kvs, vsum = _tc_b1(x_p, fc0_w, fc0_b, bn0_g, bn0_b,
                                   wq_w, wq_b, wk_w, wk_b)
    ks_sum = jnp.sum(ks, axis=0)
    ksmat = jnp.zeros((D, D), jnp.float32).at[:, 0].set(ks_sum)
    a, u0, d2 = _tc_b2(qs, x0, kvs, ksmat, vsum, hist)

    xt, ut = x0, u0
    for _ in range(2):
        acc = _sc_scatter(ut, row_p, col_p)
        xt, ut = _tc_combine(xt, acc, a, d2)
    acc = _sc_scatter(ut, row_p, col_p)
    return _tc_final(xt, acc, a, d2, x0, wo_w, wo_b, bn1_g, bn1_b,
                     fc1_w, fc1_b)
